# Initial kernel scaffold; baseline (speedup 1.0000x reference)
import jax
import jax.numpy as jnp
from jax.experimental import pallas as pl


def kernel(x, a_norm, w_self, w_neigh, sage_bias, node_biases, src, dst):
    raise NotImplementedError("write your pallas kernel here")


# speedup vs baseline: 15.3232x; 15.3232x over previous
import jax
import jax.numpy as jnp
from jax.experimental import pallas as pl
from jax.experimental.pallas import tpu as pltpu


def _sage_tables_kernel(nb_ref, a_ref, x_ref, xs_ref, w_ref, b_ref,
                        ta_ref, tb_ref):
    neigh = jnp.dot(a_ref[...], x_ref[...], preferred_element_type=jnp.float32)
    xz = jnp.concatenate([xs_ref[...], neigh], axis=1)
    h = jnp.dot(xz, w_ref[...], preferred_element_type=jnp.float32)
    h = jnp.maximum(h + b_ref[...], 0.0)

    tm = h.shape[0]
    nb = nb_ref[...]
    lane = jax.lax.broadcasted_iota(jnp.int32, (tm, 128), 1)
    zeros = jnp.zeros((tm, 128), jnp.float32)
    ones = jnp.ones((tm, 128), jnp.float32)
    ea = jnp.where(lane == 0, nb, jnp.where(lane == 1, ones, zeros))
    eb = jnp.where(lane == 0, ones, jnp.where(lane == 1, nb, zeros))
    ta_ref[...] = jnp.concatenate([h, ea], axis=1)
    tb_ref[...] = jnp.concatenate([h, eb], axis=1)


def _sage_tables(x, a_norm, w_stacked, b, nb_col, *, tm):
    n, din = x.shape
    d = w_stacked.shape[1]
    daug = d + 128

    flops = 2 * n * n * din + 2 * n * (2 * din) * d
    bytes_accessed = 4 * (n * n + n * din + 2 * din * d + d + 2 * n * daug)

    return pl.pallas_call(
        _sage_tables_kernel,
        out_shape=(jax.ShapeDtypeStruct((n, daug), jnp.float32),
                   jax.ShapeDtypeStruct((n, daug), jnp.float32)),
        grid=(n // tm,),
        in_specs=[
            pl.BlockSpec((tm, 1), lambda i: (i, 0)),
            pl.BlockSpec((tm, n), lambda i: (i, 0)),
            pl.BlockSpec((n, din), lambda i: (0, 0)),
            pl.BlockSpec((tm, din), lambda i: (i, 0)),
            pl.BlockSpec((2 * din, d), lambda i: (0, 0)),
            pl.BlockSpec((1, d), lambda i: (0, 0)),
        ],
        out_specs=(pl.BlockSpec((tm, daug), lambda i: (i, 0)),
                   pl.BlockSpec((tm, daug), lambda i: (i, 0))),
        compiler_params=pltpu.CompilerParams(
            dimension_semantics=("parallel",)),
        cost_estimate=pl.CostEstimate(flops=flops, transcendentals=0,
                                      bytes_accessed=bytes_accessed),
    )(nb_col, a_norm, x, x, w_stacked, b)


def _edge_score_kernel(sd_ref, ta_ref, tb_ref, out_ref, p_tile):
    ei = pl.program_id(0)
    te = out_ref.shape[0]
    e_total = sd_ref.shape[0] // 2
    base = ei * te

    for mi in range(te):
        s = sd_ref[base + mi]
        d = sd_ref[e_total + base + mi]
        p_tile[mi, 0] = ta_ref[s, 0] * tb_ref[d, 0]

    prod = p_tile[...]
    out_ref[...] = jnp.sum(prod, axis=2)


def _edge_scores(ta, tb, sd, *, te):
    n, _, daug = ta.shape
    e = sd.shape[0] // 2

    flops = 3 * e * daug
    bytes_accessed = 4 * (2 * n * daug + 2 * e + e)

    out = pl.pallas_call(
        _edge_score_kernel,
        out_shape=jax.ShapeDtypeStruct((e, 1), jnp.float32),
        grid_spec=pltpu.PrefetchScalarGridSpec(
            num_scalar_prefetch=1,
            grid=(e // te,),
            in_specs=[
                pl.BlockSpec((n, 1, daug), lambda ei, sd_ref: (0, 0, 0)),
                pl.BlockSpec((n, 1, daug), lambda ei, sd_ref: (0, 0, 0)),
            ],
            out_specs=pl.BlockSpec((te, 1), lambda ei, sd_ref: (ei, 0)),
            scratch_shapes=[pltpu.VMEM((te, 1, daug), jnp.float32)],
        ),
        compiler_params=pltpu.CompilerParams(
            dimension_semantics=("parallel",)),
        cost_estimate=pl.CostEstimate(flops=flops, transcendentals=0,
                                      bytes_accessed=bytes_accessed),
    )(sd, ta, tb)
    return out.reshape(e)


def kernel(x, a_norm, w_self, w_neigh, sage_bias, node_biases, src, dst):
    n, din = x.shape
    w_stacked = jnp.concatenate([w_self, w_neigh], axis=0)
    nb_col = node_biases[1:].reshape(n, 1).astype(jnp.float32)
    sd = jnp.concatenate([src, dst]).astype(jnp.int32)

    tm = 256 if n % 256 == 0 else 128
    ta, tb = _sage_tables(x, a_norm, w_stacked, sage_bias, nb_col, tm=tm)
    daug = ta.shape[1]
    ta3 = ta.reshape(n, 1, daug)
    tb3 = tb.reshape(n, 1, daug)
    return _edge_scores(ta3, tb3, sd, te=128)


# speedup vs baseline: 20.1433x; 1.3146x over previous
import jax
import jax.numpy as jnp
from jax.experimental import pallas as pl
from jax.experimental.pallas import tpu as pltpu


def _sage_tables_kernel(nb_ref, a_ref, x_ref, w_ref, b_ref, ta_ref, tb_ref):
    i = pl.program_id(0)
    tm = a_ref.shape[0]
    neigh = jnp.dot(a_ref[...], x_ref[...], preferred_element_type=jnp.float32)
    xs = x_ref[pl.ds(i * tm, tm), :]
    xz = jnp.concatenate([xs, neigh], axis=1)
    h = jnp.dot(xz, w_ref[...], preferred_element_type=jnp.float32)
    h = jnp.maximum(h + b_ref[...], 0.0)

    nb = nb_ref[...]
    lane = jax.lax.broadcasted_iota(jnp.int32, (tm, 128), 1)
    zeros = jnp.zeros((tm, 128), jnp.float32)
    ones = jnp.ones((tm, 128), jnp.float32)
    ea = jnp.where(lane == 0, nb, jnp.where(lane == 1, ones, zeros))
    eb = jnp.where(lane == 0, ones, jnp.where(lane == 1, nb, zeros))
    ta_ref[...] = jnp.concatenate([h[:, None, :], ea[:, None, :]], axis=1)
    tb_ref[...] = jnp.concatenate([h[:, None, :], eb[:, None, :]], axis=1)


def _sage_tables(x, a_norm, w_stacked, b, nb_col, *, tm):
    n, din = x.shape
    d = w_stacked.shape[1]

    flops = 2 * n * n * din + 2 * n * (2 * din) * d
    bytes_accessed = 4 * (n * n + n * din + 2 * din * d + d + 4 * n * d)

    return pl.pallas_call(
        _sage_tables_kernel,
        out_shape=(jax.ShapeDtypeStruct((n, 2, 128), jnp.float32),
                   jax.ShapeDtypeStruct((n, 2, 128), jnp.float32)),
        grid=(n // tm,),
        in_specs=[
            pl.BlockSpec((tm, 1), lambda i: (i, 0)),
            pl.BlockSpec((tm, n), lambda i: (i, 0)),
            pl.BlockSpec((n, din), lambda i: (0, 0)),
            pl.BlockSpec((2 * din, d), lambda i: (0, 0)),
            pl.BlockSpec((1, d), lambda i: (0, 0)),
        ],
        out_specs=(pl.BlockSpec((tm, 2, 128), lambda i: (i, 0, 0)),
                   pl.BlockSpec((tm, 2, 128), lambda i: (i, 0, 0))),
        compiler_params=pltpu.CompilerParams(
            dimension_semantics=("parallel",)),
        cost_estimate=pl.CostEstimate(flops=flops, transcendentals=0,
                                      bytes_accessed=bytes_accessed),
    )(nb_col, a_norm, x, w_stacked, b)


def _edge_score_kernel(sd_ref, ta_ref, tb_ref, out_ref, p_tile):
    ei = pl.program_id(0)
    te = out_ref.shape[0]
    e_total = sd_ref.shape[0] // 2
    base = ei * te

    for mi in range(te):
        s = sd_ref[base + mi]
        d = sd_ref[e_total + base + mi]
        p_tile[mi] = ta_ref[s] * tb_ref[d]

    prod = p_tile[...]
    out_ref[...] = jnp.sum(prod, axis=(1, 2)).reshape(te, 1)


def _edge_scores(ta, tb, sd, *, te):
    n = ta.shape[0]
    e = sd.shape[0] // 2

    flops = 6 * e * 128
    bytes_accessed = 4 * (4 * n * 128 + 2 * e + e)

    out = pl.pallas_call(
        _edge_score_kernel,
        out_shape=jax.ShapeDtypeStruct((e, 1), jnp.float32),
        grid_spec=pltpu.PrefetchScalarGridSpec(
            num_scalar_prefetch=1,
            grid=(e // te,),
            in_specs=[
                pl.BlockSpec((n, 2, 128), lambda ei, sd_ref: (0, 0, 0)),
                pl.BlockSpec((n, 2, 128), lambda ei, sd_ref: (0, 0, 0)),
            ],
            out_specs=pl.BlockSpec((te, 1), lambda ei, sd_ref: (ei, 0)),
            scratch_shapes=[pltpu.VMEM((te, 2, 128), jnp.float32)],
        ),
        compiler_params=pltpu.CompilerParams(
            dimension_semantics=("parallel",)),
        cost_estimate=pl.CostEstimate(flops=flops, transcendentals=0,
                                      bytes_accessed=bytes_accessed),
    )(sd, ta, tb)
    return out.reshape(e)


def kernel(x, a_norm, w_self, w_neigh, sage_bias, node_biases, src, dst):
    n, din = x.shape
    w_stacked = jnp.concatenate([w_self, w_neigh], axis=0)
    nb_col = node_biases[1:].reshape(n, 1).astype(jnp.float32)
    sd = jnp.concatenate([src, dst]).astype(jnp.int32)
    e = src.shape[0]

    tm = 512 if n % 512 == 0 else 128
    te = 512 if e % 512 == 0 else 128
    ta, tb = _sage_tables(x, a_norm, w_stacked, sage_bias, nb_col, tm=tm)
    return _edge_scores(ta, tb, sd, te=te)
